# Initial kernel scaffold; baseline (speedup 1.0000x reference)
import jax
import jax.numpy as jnp
from jax.experimental import pallas as pl


def kernel(points):
    raise NotImplementedError("write your pallas kernel here")


# speedup vs baseline: 2.4209x; 2.4209x over previous
import jax
import jax.numpy as jnp
from jax import lax
from jax.experimental import pallas as pl
from jax.experimental.pallas import tpu as pltpu
from jax.experimental.pallas import tpu_sc as plsc

B, N, F = 4, 100000, 18
FC = 24
NX = NY = 256
NV = NX * NY
DUMP = NV
ROWS = NV + 1
NC, NS, L = 2, 16, 16
CHUNK = 512
GROUPS = CHUNK // L
JROWS = CHUNK // 128
NP = 100352
PPT = NP // NS
NFULL = PPT // CHUNK
TAIL = PPT - NFULL * CHUNK
VPT = NV // NS
SUB = 512
ZROWS = 256
BPC = B // NC


def _body(points_hbm, out_hbm, accum, pts, idx, zbuf, obuf):
    c = lax.axis_index("c")
    s = lax.axis_index("s")
    iota = lax.iota(jnp.int32, L)
    zeros16 = jnp.zeros((L,), jnp.float32)

    def zrow(r, carry):
        zbuf[r, pl.ds(0, L)] = zeros16
        zbuf[r, pl.ds(FC - L, L)] = zeros16
        return carry
    lax.fori_loop(0, ZROWS, zrow, 0)

    def col(f):
        return jnp.full((L,), f, jnp.int32)

    for k in range(BPC):
        b = c * BPC + k

        for z in range(VPT // ZROWS):
            pltpu.sync_copy(zbuf, accum.at[pl.ds(s * VPT + z * ZROWS, ZROWS), :])
        plsc.subcore_barrier()

        pbase = s * PPT
        for ch in range(NFULL + 1):
            is_tail = ch == NFULL
            n = TAIL if is_tail else CHUNK
            src = points_hbm.at[b, pl.ds(pbase + ch * CHUNK, n), :]
            pltpu.sync_copy(src, pts.at[pl.ds(0, n), :] if is_tail else pts)
            ngroups = n // L

            def grp(g, carry):
                rows = g * L + iota
                px = plsc.load_gather(pts, [rows, col(0)])
                py = plsc.load_gather(pts, [rows, col(1)])
                pz = plsc.load_gather(pts, [rows, col(2)])
                valid = ((px >= -4.0) & (px <= 4.0)
                         & (py >= -4.0) & (py <= 4.0)
                         & (pz >= -4.0) & (pz <= 4.0))
                ix = jnp.clip(((px + 4.0) * 32.0).astype(jnp.int32), 0, NX - 1)
                iy = jnp.clip(((py + 4.0) * 32.0).astype(jnp.int32), 0, NY - 1)
                v = jnp.where(valid, iy * NX + ix, DUMP)
                idx[g // 8, pl.ds((g % 8) * L, L)] = v
                return carry
            lax.fori_loop(0, ngroups, grp, 0)

            for j in range(1 if is_tail else JROWS):
                pltpu.sync_copy(pts.at[pl.ds(j * 128, 128), :],
                                accum.at[idx.at[j]], add=True)
        plsc.subcore_barrier()

        for sub in range(VPT // SUB):
            v0 = s * VPT + sub * SUB
            pltpu.sync_copy(accum.at[pl.ds(v0, SUB), :],
                            pts.at[pl.ds(0, SUB), :])

            def mg(g, carry):
                rows = g * L + iota
                cnt = plsc.load_gather(pts, [rows, col(F)])
                inv = 1.0 / jnp.maximum(cnt, 1.0)
                for f in range(F):
                    val = plsc.load_gather(pts, [rows, col(f)]) * inv
                    obuf[f, pl.ds(g * L, L)] = val
                return carry
            lax.fori_loop(0, SUB // L, mg, 0)
            pltpu.sync_copy(obuf, out_hbm.at[b, :, pl.ds(v0, SUB)])
        plsc.subcore_barrier()


def kernel(points):
    pts24 = jnp.concatenate(
        [points, jnp.ones((B, N, 1), points.dtype),
         jnp.zeros((B, N, FC - F - 1), points.dtype)], axis=2)
    pts24 = jnp.pad(pts24, ((0, 0), (0, NP - N), (0, 0)),
                    constant_values=100.0)
    mesh = plsc.VectorSubcoreMesh(core_axis_name="c", subcore_axis_name="s")
    run = pl.kernel(
        _body,
        out_type=jax.ShapeDtypeStruct((B, F, NV), jnp.float32),
        mesh=mesh,
        compiler_params=pltpu.CompilerParams(use_tc_tiling_on_sc=False,
                                             needs_layout_passes=False),
        scratch_types=[
            pltpu.VMEM_SHARED((ROWS, FC), jnp.float32),
            pltpu.VMEM((CHUNK, FC), jnp.float32),
            pltpu.VMEM((JROWS, 128), jnp.int32),
            pltpu.VMEM((ZROWS, FC), jnp.float32),
            pltpu.VMEM((F, SUB), jnp.float32),
        ],
    )
    out = run(pts24)
    return out.reshape(B, F, NY, NX)


# speedup vs baseline: 3.2610x; 1.3470x over previous
import jax
import jax.numpy as jnp
from jax import lax
from jax.experimental import pallas as pl
from jax.experimental.pallas import tpu as pltpu
from jax.experimental.pallas import tpu_sc as plsc

B, N, F = 4, 100000, 18
FC = 24
CNT = F
NX = NY = 256
NV = NX * NY
DUMP = NV
ROWS = NV + 1
NC, NS, L = 2, 16, 16
CHUNK = 512
JROWS = CHUNK // 128
PPT = 6256
NFULL = 12
TAILA = PPT - NFULL * CHUNK
TAILB = N - 15 * PPT - NFULL * CHUNK
VPT = NV // NS
SUB = 256
ZROWS = 128
BPC = B // NC


def _body(points_hbm, out_hbm, accum, raw, val, idx, zbuf, obuf):
    c = lax.axis_index("c")
    s = lax.axis_index("s")
    iota = lax.iota(jnp.int32, L)
    zeros16 = jnp.zeros((L,), jnp.float32)
    cpat = jnp.where(iota == CNT - (FC - L), 1.0, 0.0)

    def col(f):
        return jnp.full((L,), f, jnp.int32)

    def zrow(r, carry):
        zbuf[r, pl.ds(0, L)] = zeros16
        zbuf[r, pl.ds(FC - L, L)] = zeros16
        return carry
    lax.fori_loop(0, ZROWS, zrow, 0)

    def prow(r, carry):
        val[r, pl.ds(FC - L, L)] = cpat
        return carry
    lax.fori_loop(0, CHUNK, prow, 0)

    for k in range(BPC):
        b = c * BPC + k

        def zcopy(z, carry):
            pltpu.sync_copy(zbuf, accum.at[pl.ds(s * VPT + z * ZROWS, ZROWS), :])
            return carry
        lax.fori_loop(0, VPT // ZROWS, zcopy, 0)
        plsc.subcore_barrier()

        pbase = s * PPT

        def do_chunk(off, n):
            pltpu.sync_copy(points_hbm.at[b, pl.ds(off * F, n * F)],
                            raw.at[pl.ds(0, n * F)])
            ngroups = 128 // L if n < CHUNK else CHUNK // L

            def grp(g, carry):
                rows = g * L + iota
                r18 = rows * F
                px = plsc.load_gather(raw, [r18])
                py = plsc.load_gather(raw, [r18 + 1])
                pz = plsc.load_gather(raw, [r18 + 2])
                valid = ((px >= -4.0) & (px <= 4.0)
                         & (py >= -4.0) & (py <= 4.0)
                         & (pz >= -4.0) & (pz <= 4.0)
                         & (rows < n))
                ix = jnp.clip(((px + 4.0) * 32.0).astype(jnp.int32), 0, NX - 1)
                iy = jnp.clip(((py + 4.0) * 32.0).astype(jnp.int32), 0, NY - 1)
                v = jnp.where(valid, iy * NX + ix, DUMP)
                idx[g // 8, pl.ds((g % 8) * L, L)] = v
                plsc.store_scatter(val, [rows, col(0)], px)
                plsc.store_scatter(val, [rows, col(1)], py)
                plsc.store_scatter(val, [rows, col(2)], pz)
                for f in range(3, F):
                    x = plsc.load_gather(raw, [r18 + f])
                    plsc.store_scatter(val, [rows, col(f)], x)
                return carry
            lax.fori_loop(0, ngroups, grp, 0)

            for j in range(1 if n < CHUNK else JROWS):
                pltpu.sync_copy(val.at[pl.ds(j * 128, 128), :],
                                accum.at[idx.at[j]], add=True)

        def chunk_loop(ch, carry):
            do_chunk(pbase + ch * CHUNK, CHUNK)
            return carry
        lax.fori_loop(0, NFULL, chunk_loop, 0)

        @pl.when(s < NS - 1)
        def _():
            do_chunk(pbase + NFULL * CHUNK, TAILA)

        @pl.when(s == NS - 1)
        def _():
            do_chunk(pbase + NFULL * CHUNK, TAILB)
        plsc.subcore_barrier()

        def sub_loop(sub, carry):
            v0 = s * VPT + sub * SUB
            pltpu.sync_copy(accum.at[pl.ds(v0, SUB), :],
                            val.at[pl.ds(0, SUB), :])

            def mg(g, carry):
                rows = g * L + iota
                cnt = plsc.load_gather(val, [rows, col(CNT)])
                inv = 1.0 / jnp.maximum(cnt, 1.0)
                for f in range(F):
                    x = plsc.load_gather(val, [rows, col(f)]) * inv
                    obuf[f, pl.ds(g * L, L)] = x
                return carry
            lax.fori_loop(0, SUB // L, mg, 0)
            pltpu.sync_copy(obuf, out_hbm.at[b, :, pl.ds(v0, SUB)])
            return carry
        lax.fori_loop(0, VPT // SUB, sub_loop, 0)

        def prow2(r, carry):
            val[r, pl.ds(FC - L, L)] = cpat
            return carry
        lax.fori_loop(0, SUB, prow2, 0)
        plsc.subcore_barrier()


def kernel(points):
    mesh = plsc.VectorSubcoreMesh(core_axis_name="c", subcore_axis_name="s")
    run = pl.kernel(
        _body,
        out_type=jax.ShapeDtypeStruct((B, F, NV), jnp.float32),
        mesh=mesh,
        compiler_params=pltpu.CompilerParams(use_tc_tiling_on_sc=False,
                                             needs_layout_passes=False),
        scratch_types=[
            pltpu.VMEM_SHARED((ROWS, FC), jnp.float32),
            pltpu.VMEM((CHUNK * F,), jnp.float32),
            pltpu.VMEM((CHUNK, FC), jnp.float32),
            pltpu.VMEM((JROWS, 128), jnp.int32),
            pltpu.VMEM((ZROWS, FC), jnp.float32),
            pltpu.VMEM((F, SUB), jnp.float32),
        ],
    )
    out = run(points.reshape(B, N * F))
    return out.reshape(B, F, NY, NX)
